# baseline (speedup 1.0000x reference)
import jax
import jax.numpy as jnp
from jax import lax
from jax.experimental import pallas as pl
from jax.experimental.pallas import tpu as pltpu

_TOPK = 13
_EPS_AM = 1e-9
_EPS_IOU = 1e-7
_HI = lax.Precision.HIGHEST


def _assigner_body(bg_ref, ps_ref, pr_ref, ap_ref, gl_ref, gr_ref, gc_ref,
                   pad_ref, lab_out, rbx_out, sco_out, gi_out, crw_out,
                   work_ref):
    n = gr_ref.shape[1]
    L, C = ps_ref.shape[1], ps_ref.shape[2]
    b = pl.program_id(0)
    f32 = jnp.float32

    ps = ps_ref[0]
    pT = pr_ref[0]
    g = gr_ref[0]
    lab = gl_ref[0]
    crw = gc_ref[0]
    pad = pad_ref[0]
    bg = bg_ref[0]

    def _t(mat):
        k = mat.shape[1]
        eye = (lax.broadcasted_iota(jnp.int32, (k, k), 0)
               == lax.broadcasted_iota(jnp.int32, (k, k), 1)).astype(f32)
        return lax.dot_general(eye, mat, (((1,), (1,)), ((), ())),
                               precision=_HI, preferred_element_type=f32)

    px, py, pw, ph, pr_ = (pT[0:1], pT[1:2], pT[2:3], pT[3:4], pT[4:5])
    apT = ap_ref[:]
    ax, ay = apT[0:1], apT[1:2]

    gx, gy, gw, gh, gr_ = (g[:, 0:1], g[:, 1:2], g[:, 2:3], g[:, 3:4], g[:, 4:5])

    def _cov(w, h, ang):
        a = w * w / 12.0
        bb = h * h / 12.0
        c = jnp.cos(ang)
        s = jnp.sin(ang)
        return a * c * c + bb * s * s, a * s * s + bb * c * c, (a - bb) * c * s

    A1, B1, C1 = _cov(gw, gh, gr_)
    A2, B2, C2 = _cov(pw, ph, pr_)
    sA = A1 + A2
    sB = B1 + B2
    sC = C1 + C2
    base = sA * sB - sC * sC
    denom = base + _EPS_IOU
    dyy = gy - py
    dxx = gx - px
    t1 = (sA * dyy * dyy + sB * dxx * dxx) / denom * 0.25
    t2 = sC * (px - gx) * dyy / denom * 0.5
    det1 = jnp.clip(A1 * B1 - C1 * C1, 0.0)
    det2 = jnp.clip(A2 * B2 - C2 * C2, 0.0)
    t3 = jnp.log(base / (4.0 * jnp.sqrt(det1 * det2 + _EPS_IOU) + _EPS_IOU)
                 + _EPS_IOU) * 0.5
    bd = jnp.clip(t1 + t2 + t3, _EPS_IOU, 10.0)
    iou = 1.0 - jnp.sqrt(1.0 - jnp.exp(-bd))

    cls_iota = lax.broadcasted_iota(jnp.int32, (n, C), 1)
    oh_lab = (lab == cls_iota).astype(f32)
    bbox_cls = lax.dot_general(oh_lab, ps, (((1,), (1,)), ((), ())),
                               precision=_HI, preferred_element_type=f32)
    i2 = iou * iou
    align = bbox_cls * (i2 * i2 * i2)

    rad = jnp.minimum(gw, gh) * 0.5
    dax = ax - gx
    day = ay - gy
    in_gts = (dax * dax + day * day <= rad * rad).astype(f32)

    metric = align * in_gts

    iota_L = lax.broadcasted_iota(jnp.int32, (n, L), 1)
    work_ref[...] = metric

    w = metric
    for _ in range(_TOPK):
        m = jnp.max(w, axis=1, keepdims=True)
        cand = jnp.where(w == m, iota_L, L)
        idx = jnp.min(cand, axis=1, keepdims=True)
        w = jnp.where(cand == idx, -1.0, w)
    work_ref[...] = w
    maskp = jnp.where(work_ref[...] < 0.0, in_gts * (pad * pad), 0.0)

    colsum = jnp.sum(maskp, axis=0, keepdims=True)
    multiple = colsum > 1.0
    iota_n = lax.broadcasted_iota(jnp.int32, (n, L), 0)
    mxio = jnp.max(iou, axis=0, keepdims=True)
    gmin = jnp.min(jnp.where(iou == mxio, iota_n, n), axis=0, keepdims=True)
    is_max = (iota_n == gmin).astype(f32)
    maskp = jnp.where(multiple, is_max, maskp)
    fg = jnp.sum(maskp, axis=0, keepdims=True) > 0.0

    gidx = jnp.min(jnp.where(maskp > 0.5, iota_n, n), axis=0, keepdims=True)
    gidx0 = jnp.where(fg, gidx, 0)
    assign_oh = (iota_n == gidx0).astype(f32)

    lab_g = jnp.sum(assign_oh * lab.astype(f32), axis=0, keepdims=True)
    labels = jnp.where(fg, lab_g.astype(jnp.int32), bg)
    crowd = jnp.sum(assign_oh * crw.astype(f32), axis=0,
                    keepdims=True).astype(jnp.int32)
    gT = _t(g)
    rbox5 = lax.dot_general(gT, assign_oh, (((1,), (0,)), ((), ())),
                            precision=_HI, preferred_element_type=f32)

    amsel = align * maskp
    maxm = jnp.max(amsel, axis=1, keepdims=True)
    maxi = jnp.max(iou * maskp, axis=1, keepdims=True)
    w_am = amsel / (maxm + _EPS_AM) * maxi
    ones_n = jnp.ones((n, 1), f32)
    am_col = lax.dot_general(w_am, ones_n, (((0,), (0,)), ((), ())),
                             precision=_HI, preferred_element_type=f32)
    lab_col = lax.dot_general(assign_oh, lab.astype(f32),
                              (((0,), (0,)), ((), ())),
                              precision=_HI, preferred_element_type=f32)
    cls_col = lax.broadcasted_iota(jnp.int32, (L, C), 1)
    scores = jnp.where(cls_col == lab_col.astype(jnp.int32), am_col, 0.0)

    lab_out[0] = labels
    gi_out[0] = gidx0 + b * n
    crw_out[0] = crowd
    rbx_out[0] = rbox5
    sco_out[0] = scores


def kernel(pred_scores, pred_rboxes, anchor_points, gt_labels, gt_rboxes,
           gt_crowd, pad_gt_mask, bg_index):
    B, L, C = pred_scores.shape
    n = gt_rboxes.shape[1]
    bg = jnp.reshape(jnp.asarray(bg_index, jnp.int32), (1,))
    prT = jnp.transpose(pred_rboxes, (0, 2, 1))
    apT = jnp.transpose(anchor_points, (1, 0))

    out = pl.pallas_call(
        _assigner_body,
        grid=(B,),
        in_specs=[
            pl.BlockSpec(memory_space=pltpu.SMEM),
            pl.BlockSpec((1, L, C), lambda b: (b, 0, 0)),
            pl.BlockSpec((1, 5, L), lambda b: (b, 0, 0)),
            pl.BlockSpec((2, L), lambda b: (0, 0)),
            pl.BlockSpec((1, n, 1), lambda b: (b, 0, 0)),
            pl.BlockSpec((1, n, 5), lambda b: (b, 0, 0)),
            pl.BlockSpec((1, n, 1), lambda b: (b, 0, 0)),
            pl.BlockSpec((1, n, 1), lambda b: (b, 0, 0)),
        ],
        out_specs=[
            pl.BlockSpec((1, 1, L), lambda b: (b, 0, 0)),
            pl.BlockSpec((1, 5, L), lambda b: (b, 0, 0)),
            pl.BlockSpec((1, L, C), lambda b: (b, 0, 0)),
            pl.BlockSpec((1, 1, L), lambda b: (b, 0, 0)),
            pl.BlockSpec((1, 1, L), lambda b: (b, 0, 0)),
        ],
        out_shape=[
            jax.ShapeDtypeStruct((B, 1, L), jnp.int32),
            jax.ShapeDtypeStruct((B, 5, L), jnp.float32),
            jax.ShapeDtypeStruct((B, L, C), jnp.float32),
            jax.ShapeDtypeStruct((B, 1, L), jnp.int32),
            jax.ShapeDtypeStruct((B, 1, L), jnp.int32),
        ],
        scratch_shapes=[
            pltpu.VMEM((n, L), jnp.float32),
        ],
    )(bg, pred_scores, prT, apT, gt_labels, gt_rboxes,
      gt_crowd, pad_gt_mask)

    labels3, rbox5, scores, gi3, crw3 = out
    return (labels3.reshape(B, L), jnp.transpose(rbox5, (0, 2, 1)), scores,
            gi3.reshape(B, L), crw3.reshape(B, L))


# speedup vs baseline: 1.1559x; 1.1559x over previous
import jax
import jax.numpy as jnp
from jax import lax
from jax.experimental import pallas as pl
from jax.experimental.pallas import tpu as pltpu

_TOPK = 13
_EPS_AM = 1e-9
_EPS_IOU = 1e-7
_HI = lax.Precision.HIGHEST


def _assigner_body(bg_ref, ps_ref, pr_ref, ap_ref, gl_ref, gr_ref, gc_ref,
                   pad_ref, lab_out, rbx_out, sco_out, gi_out, crw_out):
    n = gr_ref.shape[1]
    L, C = ps_ref.shape[1], ps_ref.shape[2]
    b = pl.program_id(0)
    f32 = jnp.float32

    ps = ps_ref[0]
    pT = pr_ref[0]
    g = gr_ref[0]
    lab = gl_ref[0]
    crw = gc_ref[0]
    pad = pad_ref[0]
    bg = bg_ref[0]

    def _t(mat):
        k = mat.shape[1]
        eye = (lax.broadcasted_iota(jnp.int32, (k, k), 0)
               == lax.broadcasted_iota(jnp.int32, (k, k), 1)).astype(f32)
        return lax.dot_general(eye, mat, (((1,), (1,)), ((), ())),
                               precision=_HI, preferred_element_type=f32)

    px, py, pw, ph, pr_ = (pT[0:1], pT[1:2], pT[2:3], pT[3:4], pT[4:5])
    apT = ap_ref[:]
    ax, ay = apT[0:1], apT[1:2]

    gx, gy, gw, gh, gr_ = (g[:, 0:1], g[:, 1:2], g[:, 2:3], g[:, 3:4], g[:, 4:5])

    def _cov(w, h, ang):
        a = w * w / 12.0
        bb = h * h / 12.0
        c = jnp.cos(ang)
        s = jnp.sin(ang)
        return a * c * c + bb * s * s, a * s * s + bb * c * c, (a - bb) * c * s

    A1, B1, C1 = _cov(gw, gh, gr_)
    A2, B2, C2 = _cov(pw, ph, pr_)
    sA = A1 + A2
    sB = B1 + B2
    sC = C1 + C2
    base = sA * sB - sC * sC
    denom = base + _EPS_IOU
    dyy = gy - py
    dxx = gx - px
    t1 = (sA * dyy * dyy + sB * dxx * dxx) / denom * 0.25
    t2 = sC * (px - gx) * dyy / denom * 0.5
    det1 = jnp.clip(A1 * B1 - C1 * C1, 0.0)
    det2 = jnp.clip(A2 * B2 - C2 * C2, 0.0)
    t3 = jnp.log(base / (4.0 * jnp.sqrt(det1 * det2 + _EPS_IOU) + _EPS_IOU)
                 + _EPS_IOU) * 0.5
    bd = jnp.clip(t1 + t2 + t3, _EPS_IOU, 10.0)
    iou = 1.0 - jnp.sqrt(1.0 - jnp.exp(-bd))

    cls_iota = lax.broadcasted_iota(jnp.int32, (n, C), 1)
    oh_lab = (lab == cls_iota).astype(f32)
    bbox_cls = lax.dot_general(oh_lab, ps, (((1,), (1,)), ((), ())),
                               precision=_HI, preferred_element_type=f32)
    i2 = iou * iou
    align = bbox_cls * (i2 * i2 * i2)

    rad = jnp.minimum(gw, gh) * 0.5
    dax = ax - gx
    day = ay - gy
    in_gts = (dax * dax + day * day <= rad * rad).astype(f32)

    metric = align * in_gts

    iota_L = lax.broadcasted_iota(jnp.int32, (n, L), 1)
    w = metric
    for _ in range(_TOPK):
        m = jnp.max(w, axis=1, keepdims=True)
        cand = jnp.where(w == m, iota_L, L)
        idx = jnp.min(cand, axis=1, keepdims=True)
        w = jnp.where(cand == idx, -1.0, w)
    maskp = jnp.where(w < 0.0, in_gts * (pad * pad), 0.0)

    colsum = jnp.sum(maskp, axis=0, keepdims=True)
    multiple = colsum > 1.0
    iota_n = lax.broadcasted_iota(jnp.int32, (n, L), 0)
    mxio = jnp.max(iou, axis=0, keepdims=True)
    gmin = jnp.min(jnp.where(iou == mxio, iota_n, n), axis=0, keepdims=True)
    is_max = (iota_n == gmin).astype(f32)
    maskp = jnp.where(multiple, is_max, maskp)
    fg = jnp.sum(maskp, axis=0, keepdims=True) > 0.0

    gidx = jnp.min(jnp.where(maskp > 0.5, iota_n, n), axis=0, keepdims=True)
    gidx0 = jnp.where(fg, gidx, 0)
    assign_oh = (iota_n == gidx0).astype(f32)

    packed = lab.astype(f32) + 128.0 * crw.astype(f32)
    pk = jnp.sum(assign_oh * packed, axis=0, keepdims=True)
    crowd = (pk >= 128.0).astype(jnp.int32)
    lab_g = pk - 128.0 * crowd.astype(f32)
    labels = jnp.where(fg, lab_g.astype(jnp.int32), bg)
    gT = _t(g)
    rbox5 = lax.dot_general(gT, assign_oh, (((1,), (0,)), ((), ())),
                            precision=_HI, preferred_element_type=f32)

    amsel = align * maskp
    maxm = jnp.max(amsel, axis=1, keepdims=True)
    maxi = jnp.max(iou * maskp, axis=1, keepdims=True)
    w_am = amsel / (maxm + _EPS_AM) * maxi
    scores = lax.dot_general(w_am, oh_lab, (((0,), (0,)), ((), ())),
                             precision=_HI, preferred_element_type=f32)

    lab_out[0] = labels
    gi_out[0] = gidx0 + b * n
    crw_out[0] = crowd
    rbx_out[0] = rbox5
    sco_out[0] = scores


def kernel(pred_scores, pred_rboxes, anchor_points, gt_labels, gt_rboxes,
           gt_crowd, pad_gt_mask, bg_index):
    B, L, C = pred_scores.shape
    n = gt_rboxes.shape[1]
    bg = jnp.reshape(jnp.asarray(bg_index, jnp.int32), (1,))
    prT = jnp.transpose(pred_rboxes, (0, 2, 1))
    apT = jnp.transpose(anchor_points, (1, 0))

    out = pl.pallas_call(
        _assigner_body,
        grid=(B,),
        in_specs=[
            pl.BlockSpec(memory_space=pltpu.SMEM),
            pl.BlockSpec((1, L, C), lambda b: (b, 0, 0)),
            pl.BlockSpec((1, 5, L), lambda b: (b, 0, 0)),
            pl.BlockSpec((2, L), lambda b: (0, 0)),
            pl.BlockSpec((1, n, 1), lambda b: (b, 0, 0)),
            pl.BlockSpec((1, n, 5), lambda b: (b, 0, 0)),
            pl.BlockSpec((1, n, 1), lambda b: (b, 0, 0)),
            pl.BlockSpec((1, n, 1), lambda b: (b, 0, 0)),
        ],
        out_specs=[
            pl.BlockSpec((1, 1, L), lambda b: (b, 0, 0)),
            pl.BlockSpec((1, 5, L), lambda b: (b, 0, 0)),
            pl.BlockSpec((1, L, C), lambda b: (b, 0, 0)),
            pl.BlockSpec((1, 1, L), lambda b: (b, 0, 0)),
            pl.BlockSpec((1, 1, L), lambda b: (b, 0, 0)),
        ],
        out_shape=[
            jax.ShapeDtypeStruct((B, 1, L), jnp.int32),
            jax.ShapeDtypeStruct((B, 5, L), jnp.float32),
            jax.ShapeDtypeStruct((B, L, C), jnp.float32),
            jax.ShapeDtypeStruct((B, 1, L), jnp.int32),
            jax.ShapeDtypeStruct((B, 1, L), jnp.int32),
        ],
    )(bg, pred_scores, prT, apT, gt_labels, gt_rboxes,
      gt_crowd, pad_gt_mask)

    labels3, rbox5, scores, gi3, crw3 = out
    return (labels3.reshape(B, L), jnp.transpose(rbox5, (0, 2, 1)), scores,
            gi3.reshape(B, L), crw3.reshape(B, L))


# speedup vs baseline: 1.7289x; 1.4957x over previous
import jax
import jax.numpy as jnp
from jax import lax
from jax.experimental import pallas as pl
from jax.experimental.pallas import tpu as pltpu

_TOPK = 13
_EPS_AM = 1e-9
_EPS_IOU = 1e-7
_HI = lax.Precision.HIGHEST


def _assigner_body(bg_ref, ps_ref, pr_ref, ap_ref, gl_ref, gr_ref, gc_ref,
                   pad_ref, lab_out, rbx_out, sco_out, gi_out, crw_out):
    n = gr_ref.shape[1]
    C, L = ps_ref.shape[1], ps_ref.shape[2]
    b = pl.program_id(0)
    f32 = jnp.float32

    psT = ps_ref[0]
    pT = pr_ref[0]
    g = gr_ref[0]
    lab = gl_ref[0]
    crw = gc_ref[0]
    pad = pad_ref[0]
    bg = bg_ref[0]

    def _t(mat):
        k = mat.shape[1]
        eye = (lax.broadcasted_iota(jnp.int32, (k, k), 0)
               == lax.broadcasted_iota(jnp.int32, (k, k), 1)).astype(f32)
        return lax.dot_general(eye, mat, (((1,), (1,)), ((), ())),
                               precision=_HI, preferred_element_type=f32)

    px, py, pw, ph, pr_ = (pT[0:1], pT[1:2], pT[2:3], pT[3:4], pT[4:5])
    apT = ap_ref[:]
    ax, ay = apT[0:1], apT[1:2]

    gx, gy, gw, gh, gr_ = (g[:, 0:1], g[:, 1:2], g[:, 2:3], g[:, 3:4], g[:, 4:5])

    def _cov(w, h, ang):
        a = w * w / 12.0
        bb = h * h / 12.0
        c = jnp.cos(ang)
        s = jnp.sin(ang)
        return a * c * c + bb * s * s, a * s * s + bb * c * c, (a - bb) * c * s

    A1, B1, C1 = _cov(gw, gh, gr_)
    A2, B2, C2 = _cov(pw, ph, pr_)
    sA = A1 + A2
    sB = B1 + B2
    sC = C1 + C2
    base = sA * sB - sC * sC
    denom = base + _EPS_IOU
    dyy = gy - py
    dxx = gx - px
    t1 = (sA * dyy * dyy + sB * dxx * dxx) / denom * 0.25
    t2 = sC * (px - gx) * dyy / denom * 0.5
    det1 = jnp.clip(A1 * B1 - C1 * C1, 0.0)
    det2 = jnp.clip(A2 * B2 - C2 * C2, 0.0)
    t3 = jnp.log(base / (4.0 * jnp.sqrt(det1 * det2 + _EPS_IOU) + _EPS_IOU)
                 + _EPS_IOU) * 0.5
    bd = jnp.clip(t1 + t2 + t3, _EPS_IOU, 10.0)
    iou = 1.0 - jnp.sqrt(1.0 - jnp.exp(-bd))

    cls_iota = lax.broadcasted_iota(jnp.int32, (n, C), 1)
    oh_lab = (lab == cls_iota).astype(f32)
    bbox_cls = lax.dot_general(oh_lab, psT, (((1,), (0,)), ((), ())),
                               precision=_HI, preferred_element_type=f32)
    i2 = iou * iou
    align = bbox_cls * (i2 * i2 * i2)

    rad = jnp.minimum(gw, gh) * 0.5
    dax = ax - gx
    day = ay - gy
    in_gts = (dax * dax + day * day <= rad * rad).astype(f32)

    metric = align * in_gts

    iota_L = lax.broadcasted_iota(jnp.int32, (n, L), 1)
    w = metric
    for _ in range(_TOPK):
        m = jnp.max(w, axis=1, keepdims=True)
        cand = jnp.where(w == m, iota_L, L)
        idx = jnp.min(cand, axis=1, keepdims=True)
        w = jnp.where(cand == idx, -1.0, w)
    maskp = jnp.where(w < 0.0, in_gts * (pad * pad), 0.0)

    colsum = jnp.sum(maskp, axis=0, keepdims=True)
    multiple = colsum > 1.0
    iota_n = lax.broadcasted_iota(jnp.int32, (n, L), 0)
    mxio = jnp.max(iou, axis=0, keepdims=True)
    gmin = jnp.min(jnp.where(iou == mxio, iota_n, n), axis=0, keepdims=True)
    is_max = (iota_n == gmin).astype(f32)
    maskp = jnp.where(multiple, is_max, maskp)
    fg = jnp.sum(maskp, axis=0, keepdims=True) > 0.0

    gidx = jnp.min(jnp.where(maskp > 0.5, iota_n, n), axis=0, keepdims=True)
    gidx0 = jnp.where(fg, gidx, 0)
    assign_oh = (iota_n == gidx0).astype(f32)

    packed = lab.astype(f32) + 128.0 * crw.astype(f32)
    pk = jnp.sum(assign_oh * packed, axis=0, keepdims=True)
    crowd = (pk >= 128.0).astype(jnp.int32)
    lab_g = pk - 128.0 * crowd.astype(f32)
    labels = jnp.where(fg, lab_g.astype(jnp.int32), bg)
    gT = _t(g)
    rbox5 = lax.dot_general(gT, assign_oh, (((1,), (0,)), ((), ())),
                            precision=_HI, preferred_element_type=f32)

    amsel = align * maskp
    maxm = jnp.max(amsel, axis=1, keepdims=True)
    maxi = jnp.max(iou * maskp, axis=1, keepdims=True)
    w_am = amsel / (maxm + _EPS_AM) * maxi
    scores = lax.dot_general(oh_lab, w_am, (((0,), (0,)), ((), ())),
                             precision=_HI, preferred_element_type=f32)

    lab_out[0] = labels
    gi_out[0] = gidx0 + b * n
    crw_out[0] = crowd
    rbx_out[0] = rbox5
    sco_out[0] = scores


def kernel(pred_scores, pred_rboxes, anchor_points, gt_labels, gt_rboxes,
           gt_crowd, pad_gt_mask, bg_index):
    B, L, C = pred_scores.shape
    n = gt_rboxes.shape[1]
    bg = jnp.reshape(jnp.asarray(bg_index, jnp.int32), (1,))
    psT = jnp.transpose(pred_scores, (0, 2, 1))
    prT = jnp.transpose(pred_rboxes, (0, 2, 1))
    apT = jnp.transpose(anchor_points, (1, 0))

    out = pl.pallas_call(
        _assigner_body,
        grid=(B,),
        in_specs=[
            pl.BlockSpec(memory_space=pltpu.SMEM),
            pl.BlockSpec((1, C, L), lambda b: (b, 0, 0)),
            pl.BlockSpec((1, 5, L), lambda b: (b, 0, 0)),
            pl.BlockSpec((2, L), lambda b: (0, 0)),
            pl.BlockSpec((1, n, 1), lambda b: (b, 0, 0)),
            pl.BlockSpec((1, n, 5), lambda b: (b, 0, 0)),
            pl.BlockSpec((1, n, 1), lambda b: (b, 0, 0)),
            pl.BlockSpec((1, n, 1), lambda b: (b, 0, 0)),
        ],
        out_specs=[
            pl.BlockSpec((1, 1, L), lambda b: (b, 0, 0)),
            pl.BlockSpec((1, 5, L), lambda b: (b, 0, 0)),
            pl.BlockSpec((1, C, L), lambda b: (b, 0, 0)),
            pl.BlockSpec((1, 1, L), lambda b: (b, 0, 0)),
            pl.BlockSpec((1, 1, L), lambda b: (b, 0, 0)),
        ],
        out_shape=[
            jax.ShapeDtypeStruct((B, 1, L), jnp.int32),
            jax.ShapeDtypeStruct((B, 5, L), jnp.float32),
            jax.ShapeDtypeStruct((B, C, L), jnp.float32),
            jax.ShapeDtypeStruct((B, 1, L), jnp.int32),
            jax.ShapeDtypeStruct((B, 1, L), jnp.int32),
        ],
    )(bg, psT, prT, apT, gt_labels, gt_rboxes,
      gt_crowd, pad_gt_mask)

    labels3, rbox5, scoresT, gi3, crw3 = out
    return (labels3.reshape(B, L), jnp.transpose(rbox5, (0, 2, 1)),
            jnp.transpose(scoresT, (0, 2, 1)),
            gi3.reshape(B, L), crw3.reshape(B, L))
